# initial kernel scaffold (unmeasured)
import jax
import jax.numpy as jnp
from jax import lax
from jax.experimental import pallas as pl
from jax.experimental.pallas import tpu as pltpu

N_DEV = 4


def kernel(x, Win0, Wout0, Win1, Wout1, Win2, Wout2):
    m_per, d = x.shape
    m = N_DEV * m_per

    def body(x_ref, win0_ref, wout0_ref, win1_ref, wout1_ref, win2_ref,
             wout2_ref, out_ref, xfull, partials,
             x_send_sems, x_recv_sems, p_send_sems, p_recv_sems, credit_sem):
        me = lax.axis_index("i")
        left = (me - 1) % N_DEV
        right = (me + 1) % N_DEV

        barrier_sem = pltpu.get_barrier_semaphore()
        for nbr in (left, right):
            pl.semaphore_signal(
                barrier_sem, inc=1,
                device_id=(nbr,), device_id_type=pl.DeviceIdType.MESH,
            )
        pl.semaphore_wait(barrier_sem, 2)

        xfull[pl.ds(me * m_per, m_per), :] = x_ref[:, :]
        for h in range(N_DEV - 1):
            origin = (me - h) % N_DEV
            rdma = pltpu.make_async_remote_copy(
                src_ref=xfull.at[pl.ds(origin * m_per, m_per), :],
                dst_ref=xfull.at[pl.ds(origin * m_per, m_per), :],
                send_sem=x_send_sems.at[h],
                recv_sem=x_recv_sems.at[h],
                device_id=(right,),
                device_id_type=pl.DeviceIdType.MESH,
            )
            rdma.start()
            rdma.wait()

        xin = xfull[:, :]
        for l, (win, wout) in enumerate(
            [(win0_ref, wout0_ref), (win1_ref, wout1_ref), (win2_ref, wout2_ref)]
        ):
            hid = jnp.maximum(
                jnp.dot(xin, win[:, :], preferred_element_type=jnp.float32), 0.0
            )
            part = jnp.dot(hid, wout[:, :], preferred_element_type=jnp.float32)

            if l > 0:
                pl.semaphore_wait(credit_sem, 1)

            partials[pl.ds(me, 1), :, :] = part[None]
            for h in range(N_DEV - 1):
                origin = (me - h) % N_DEV
                rdma = pltpu.make_async_remote_copy(
                    src_ref=partials.at[pl.ds(origin, 1), :, :],
                    dst_ref=partials.at[pl.ds(origin, 1), :, :],
                    send_sem=p_send_sems.at[h],
                    recv_sem=p_recv_sems.at[h],
                    device_id=(right,),
                    device_id_type=pl.DeviceIdType.MESH,
                )
                rdma.start()
                rdma.wait()

            xin = (partials[0, :, :] + partials[1, :, :]
                   + partials[2, :, :] + partials[3, :, :])
            pl.semaphore_signal(
                credit_sem, inc=1,
                device_id=(left,), device_id_type=pl.DeviceIdType.MESH,
            )

        out_ref[:, :] = xin
        pl.semaphore_wait(credit_sem, 1)

    return pl.pallas_call(
        body,
        out_shape=jax.ShapeDtypeStruct((m, d), jnp.float32),
        in_specs=[pl.BlockSpec(memory_space=pltpu.VMEM)] * 7,
        out_specs=pl.BlockSpec(memory_space=pltpu.VMEM),
        scratch_shapes=[
            pltpu.VMEM((m, d), jnp.float32),
            pltpu.VMEM((N_DEV, m, d), jnp.float32),
            pltpu.SemaphoreType.DMA((N_DEV - 1,)),
            pltpu.SemaphoreType.DMA((N_DEV - 1,)),
            pltpu.SemaphoreType.DMA((N_DEV - 1,)),
            pltpu.SemaphoreType.DMA((N_DEV - 1,)),
            pltpu.SemaphoreType.REGULAR,
        ],
        compiler_params=pltpu.CompilerParams(collective_id=0),
    )(x, Win0, Wout0, Win1, Wout1, Win2, Wout2)


# baseline (device time: 161525 ns/iter reference)
import jax
import jax.numpy as jnp
from jax import lax
from jax.experimental import pallas as pl
from jax.experimental.pallas import tpu as pltpu

N_DEV = 4


def kernel(x, Win0, Wout0, Win1, Wout1, Win2, Wout2):
    m_per, d = x.shape
    m = N_DEV * m_per

    def body(x_ref, win0_ref, wout0_ref, win1_ref, wout1_ref, win2_ref,
             wout2_ref, out_ref, xfull, partials,
             x_send_sems, x_recv_sems, p_send_sems, p_recv_sems, credit_sem):
        me = lax.axis_index("i")
        left = (me - 1) % N_DEV
        right = (me + 1) % N_DEV

        barrier_sem = pltpu.get_barrier_semaphore()
        for nbr in (left, right):
            pl.semaphore_signal(
                barrier_sem, inc=1,
                device_id=(nbr,), device_id_type=pl.DeviceIdType.MESH,
            )
        pl.semaphore_wait(barrier_sem, 2)

        xfull[pl.ds(me * m_per, m_per), :] = x_ref[:, :]
        for h in range(N_DEV - 1):
            origin = (me - h) % N_DEV
            rdma = pltpu.make_async_remote_copy(
                src_ref=xfull.at[pl.ds(origin * m_per, m_per), :],
                dst_ref=xfull.at[pl.ds(origin * m_per, m_per), :],
                send_sem=x_send_sems.at[h],
                recv_sem=x_recv_sems.at[h],
                device_id=(right,),
                device_id_type=pl.DeviceIdType.MESH,
            )
            rdma.start()
            rdma.wait()

        xin = xfull[:, :]
        for l, (win, wout) in enumerate(
            [(win0_ref, wout0_ref), (win1_ref, wout1_ref), (win2_ref, wout2_ref)]
        ):
            hid = jnp.maximum(
                jnp.dot(xin, win[:, :], preferred_element_type=jnp.float32), 0.0
            )
            part = jnp.dot(hid, wout[:, :], preferred_element_type=jnp.float32)

            if l > 0:
                pl.semaphore_wait(credit_sem, 1)

            partials[pl.ds(me, 1), :, :] = part[None]
            for h in range(N_DEV - 1):
                origin = (me - h) % N_DEV
                rdma = pltpu.make_async_remote_copy(
                    src_ref=partials.at[pl.ds(origin, 1), :, :],
                    dst_ref=partials.at[pl.ds(origin, 1), :, :],
                    send_sem=p_send_sems.at[h],
                    recv_sem=p_recv_sems.at[h],
                    device_id=(right,),
                    device_id_type=pl.DeviceIdType.MESH,
                )
                rdma.start()
                rdma.wait()

            xin = (partials[0, :, :] + partials[1, :, :]
                   + partials[2, :, :] + partials[3, :, :])
            pl.semaphore_signal(
                credit_sem, inc=1,
                device_id=(left,), device_id_type=pl.DeviceIdType.MESH,
            )

        out_ref[:, :] = xin
        pl.semaphore_wait(credit_sem, 1)

    return pl.pallas_call(
        body,
        out_shape=jax.ShapeDtypeStruct((m, d), jnp.float32),
        in_specs=[pl.BlockSpec(memory_space=pltpu.VMEM)] * 7,
        out_specs=pl.BlockSpec(memory_space=pltpu.VMEM),
        scratch_shapes=[
            pltpu.VMEM((m, d), jnp.float32),
            pltpu.VMEM((N_DEV, m, d), jnp.float32),
            pltpu.SemaphoreType.DMA((N_DEV - 1,)),
            pltpu.SemaphoreType.DMA((N_DEV - 1,)),
            pltpu.SemaphoreType.DMA((N_DEV - 1,)),
            pltpu.SemaphoreType.DMA((N_DEV - 1,)),
            pltpu.SemaphoreType.REGULAR,
        ],
        compiler_params=pltpu.CompilerParams(
            collective_id=0, vmem_limit_bytes=100 * 1024 * 1024
        ),
    )(x, Win0, Wout0, Win1, Wout1, Win2, Wout2)


# device time: 84409 ns/iter; 1.9136x vs baseline; 1.9136x over previous
import jax
import jax.numpy as jnp
from jax import lax
from jax.experimental import pallas as pl
from jax.experimental.pallas import tpu as pltpu

N_DEV = 4
N_PHASES = 1 + 2 * 3


def kernel(x, Win0, Wout0, Win1, Wout1, Win2, Wout2):
    m_per, d = x.shape
    m = N_DEV * m_per

    def body(x_ref, win0_ref, wout0_ref, win1_ref, wout1_ref, win2_ref,
             wout2_ref, out_ref, xfull, xmid, psend, rs_buf,
             send_sems, recv_sems):
        me = lax.axis_index("i")

        barrier_sem = pltpu.get_barrier_semaphore()
        for o in range(1, N_DEV):
            pl.semaphore_signal(
                barrier_sem, inc=1,
                device_id=((me + o) % N_DEV,),
                device_id_type=pl.DeviceIdType.MESH,
            )
        pl.semaphore_wait(barrier_sem, N_DEV - 1)

        def exchange_sends(phase, src_ref, dst_ref, row_of_sender):
            rdmas = []
            for o in range(1, N_DEV):
                rdma = pltpu.make_async_remote_copy(
                    src_ref=src_ref.at[pl.ds(row_of_sender * m_per, m_per), :],
                    dst_ref=dst_ref.at[pl.ds(row_of_sender * m_per, m_per), :],
                    send_sem=send_sems.at[phase, o - 1],
                    recv_sem=recv_sems.at[phase, o - 1],
                    device_id=((me + o) % N_DEV,),
                    device_id_type=pl.DeviceIdType.MESH,
                )
                rdma.start()
                rdmas.append(rdma)
            return rdmas

        def wait_recvs(phase, slice_for_offset):
            for o in range(1, N_DEV):
                dst = slice_for_offset(o)
                rdma = pltpu.make_async_remote_copy(
                    src_ref=dst,
                    dst_ref=dst,
                    send_sem=send_sems.at[phase, o - 1],
                    recv_sem=recv_sems.at[phase, o - 1],
                    device_id=((me + o) % N_DEV,),
                    device_id_type=pl.DeviceIdType.MESH,
                )
                rdma.wait_recv()

        xfull[pl.ds(me * m_per, m_per), :] = x_ref[:, :]
        ag_rdmas = exchange_sends(0, xfull, xfull, me)
        for r in ag_rdmas:
            r.wait_send()
        wait_recvs(
            0,
            lambda o: xfull.at[pl.ds(((me - o) % N_DEV) * m_per, m_per), :],
        )

        layers = [
            (win0_ref, wout0_ref, xmid.at[0]),
            (win1_ref, wout1_ref, xmid.at[1]),
            (win2_ref, wout2_ref, out_ref),
        ]
        xin_ref = xfull
        for l, (win, wout, dst) in enumerate(layers):
            rs_phase, ag_phase = 1 + 2 * l, 2 + 2 * l

            hid = jnp.maximum(
                jnp.dot(xin_ref[:, :], win[:, :],
                        preferred_element_type=jnp.float32),
                0.0,
            )
            part = jnp.dot(hid, wout[:, :], preferred_element_type=jnp.float32)
            psend[:, :] = part

            rs_rdmas = []
            for o in range(1, N_DEV):
                rdma = pltpu.make_async_remote_copy(
                    src_ref=psend.at[pl.ds(((me + o) % N_DEV) * m_per, m_per), :],
                    dst_ref=rs_buf.at[l, o - 1],
                    send_sem=send_sems.at[rs_phase, o - 1],
                    recv_sem=recv_sems.at[rs_phase, o - 1],
                    device_id=((me + o) % N_DEV,),
                    device_id_type=pl.DeviceIdType.MESH,
                )
                rdma.start()
                rs_rdmas.append(rdma)
            for r in rs_rdmas:
                r.wait_send()
            wait_recvs(rs_phase, lambda o: rs_buf.at[l, o - 1])

            red = (psend[pl.ds(me * m_per, m_per), :]
                   + rs_buf[l, 0] + rs_buf[l, 1] + rs_buf[l, 2])
            dst[pl.ds(me * m_per, m_per), :] = red

            ag_rdmas = exchange_sends(ag_phase, dst, dst, me)
            for r in ag_rdmas:
                r.wait_send()
            wait_recvs(
                ag_phase,
                lambda o: dst.at[pl.ds(((me - o) % N_DEV) * m_per, m_per), :],
            )

            xin_ref = dst

    return pl.pallas_call(
        body,
        out_shape=jax.ShapeDtypeStruct((m, d), jnp.float32),
        in_specs=[pl.BlockSpec(memory_space=pltpu.VMEM)] * 7,
        out_specs=pl.BlockSpec(memory_space=pltpu.VMEM),
        scratch_shapes=[
            pltpu.VMEM((m, d), jnp.float32),
            pltpu.VMEM((2, m, d), jnp.float32),
            pltpu.VMEM((m, d), jnp.float32),
            pltpu.VMEM((3, N_DEV - 1, m_per, d), jnp.float32),
            pltpu.SemaphoreType.DMA((N_PHASES, N_DEV - 1)),
            pltpu.SemaphoreType.DMA((N_PHASES, N_DEV - 1)),
        ],
        compiler_params=pltpu.CompilerParams(
            collective_id=0, vmem_limit_bytes=100 * 1024 * 1024
        ),
    )(x, Win0, Wout0, Win1, Wout1, Win2, Wout2)
